# baseline (speedup 1.0000x reference)
import functools

import jax
import jax.numpy as jnp
from jax import lax
from jax.experimental import pallas as pl
from jax.experimental.pallas import tpu as pltpu
from jax.experimental.pallas import tpu_sc as plsc

NUM_ROWS = 1_000_000
D = 16
B = 16384
L = 20
BL = B * L
INT_WEIGHT = 0.1
POP_WEIGHT = 0.1
DIS_PEN = 0.1

NC = 2
NS = 16
NW = NC * NS

HALF = NUM_ROWS // NC
SLAB = 31264
CNT_SZ = NS * SLAB
DUMP = HALF

PER_W = BL // NW
CHUNK = 128
N_CH = PER_W // CHUNK

IDX_BATCH = 4096
N_SCAT = IDX_BATCH // CHUNK
PER_S = BL // NS
N_BATCH = PER_S // IDX_BATCH


def _fill_const(ref, n, value, dtype):
    vec = jnp.full((16,), value, dtype=dtype)
    for k in range(n // 16):
        ref[pl.ds(k * 16, 16)] = vec


def _hist_body(itemp_hbm, itemn_hbm, user_hbm, cnt_i_out, cnt_u_out,
               idx_v, tmp_v, ones_v, zer_v, cnt_i_sp, cnt_u_sp):
    cid = lax.axis_index("c")
    sid = lax.axis_index("s")
    lo = cid * HALF

    _fill_const(ones_v, CHUNK, 1.0, jnp.float32)

    def zero_body(i, _):
        zer_v[pl.ds(i * 16, 16)] = jnp.zeros((16,), jnp.float32)
        return 0

    lax.fori_loop(0, SLAB // 16, zero_body, 0)
    pltpu.sync_copy(zer_v, cnt_i_sp.at[pl.ds(sid * SLAB, SLAB)])
    pltpu.sync_copy(zer_v, cnt_u_sp.at[pl.ds(sid * SLAB, SLAB)])
    plsc.subcore_barrier()

    def scatter_all(src_hbm, cnt_sp):
        base = sid * PER_S

        def batch_body(bi, _):
            pltpu.sync_copy(src_hbm.at[pl.ds(base + bi * IDX_BATCH, IDX_BATCH)],
                            idx_v)

            def scat_body(j, _):
                for k in range(CHUNK // 16):
                    v = idx_v[pl.ds(j * CHUNK + k * 16, 16)]
                    loc = v - lo
                    ok = (loc >= 0) & (loc < HALF)
                    tmp_v[pl.ds(k * 16, 16)] = jnp.where(ok, loc, DUMP)
                pltpu.sync_copy(ones_v, cnt_sp.at[tmp_v], add=True)
                return 0

            lax.fori_loop(0, N_SCAT, scat_body, 0)
            return 0

        lax.fori_loop(0, N_BATCH, batch_body, 0)

    scatter_all(itemp_hbm, cnt_i_sp)
    scatter_all(itemn_hbm, cnt_i_sp)
    scatter_all(user_hbm, cnt_u_sp)
    plsc.subcore_barrier()

    pltpu.sync_copy(cnt_i_sp.at[pl.ds(sid * SLAB, SLAB)], zer_v)
    pltpu.sync_copy(zer_v, cnt_i_out.at[pl.ds(cid * CNT_SZ + sid * SLAB, SLAB)])
    pltpu.sync_copy(cnt_u_sp.at[pl.ds(sid * SLAB, SLAB)], zer_v)
    pltpu.sync_copy(zer_v, cnt_u_out.at[pl.ds(cid * CNT_SZ + sid * SLAB, SLAB)])


_hist = pl.kernel(
    _hist_body,
    out_type=(jax.ShapeDtypeStruct((NC * CNT_SZ,), jnp.float32),
              jax.ShapeDtypeStruct((NC * CNT_SZ,), jnp.float32)),
    mesh=plsc.VectorSubcoreMesh(core_axis_name="c", subcore_axis_name="s"),
    compiler_params=pltpu.CompilerParams(needs_layout_passes=False, use_tc_tiling_on_sc=False),
    scratch_types=[
        pltpu.VMEM((IDX_BATCH,), jnp.int32),
        pltpu.VMEM((CHUNK,), jnp.int32),
        pltpu.VMEM((CHUNK,), jnp.float32),
        pltpu.VMEM((SLAB,), jnp.float32),
        pltpu.VMEM_SHARED((CNT_SZ,), jnp.float32),
        pltpu.VMEM_SHARED((CNT_SZ,), jnp.float32),
    ],
)


def _score_body(user_hbm, itemp_hbm, itemn_hbm,
                u_int_hbm, u_pop_hbm, i_int_hbm, i_pop_hbm,
                o_pi_hbm, o_ni_hbm, o_pp_hbm, o_np_hbm,
                iu_v, ip_v, in_v,
                ru_i, ru_p, rp_i, rp_p, rn_i, rn_p,
                o_pi, o_ni, o_pp, o_np, sem):
    cid = lax.axis_index("c")
    sid = lax.axis_index("s")
    wid = sid * NC + cid
    base = wid * PER_W

    def chunk_body(g, _):
        off = base + g * CHUNK
        pltpu.sync_copy(user_hbm.at[pl.ds(off, CHUNK)], iu_v)
        pltpu.sync_copy(itemp_hbm.at[pl.ds(off, CHUNK)], ip_v)
        pltpu.sync_copy(itemn_hbm.at[pl.ds(off, CHUNK)], in_v)
        cps = [
            pltpu.async_copy(u_int_hbm.at[iu_v], ru_i, sem),
            pltpu.async_copy(u_pop_hbm.at[iu_v], ru_p, sem),
            pltpu.async_copy(i_int_hbm.at[ip_v], rp_i, sem),
            pltpu.async_copy(i_pop_hbm.at[ip_v], rp_p, sem),
            pltpu.async_copy(i_int_hbm.at[in_v], rn_i, sem),
            pltpu.async_copy(i_pop_hbm.at[in_v], rn_p, sem),
        ]
        for cp in cps:
            cp.wait()

        lanes = lax.iota(jnp.int32, 16)
        for grp in range(CHUNK // 16):
            v_pi = jnp.zeros((16,), jnp.float32)
            v_ni = jnp.zeros((16,), jnp.float32)
            v_pp = jnp.zeros((16,), jnp.float32)
            v_np = jnp.zeros((16,), jnp.float32)
            for r16 in range(16):
                r = grp * 16 + r16
                m = lanes == r16
                ui = ru_i[r, :]
                up = ru_p[r, :]
                v_pi = jnp.where(m, jnp.sum(ui * rp_i[r, :]), v_pi)
                v_ni = jnp.where(m, jnp.sum(ui * rn_i[r, :]), v_ni)
                v_pp = jnp.where(m, jnp.sum(up * rp_p[r, :]), v_pp)
                v_np = jnp.where(m, jnp.sum(up * rn_p[r, :]), v_np)
            dst = pl.ds(g * CHUNK + grp * 16, 16)
            o_pi[dst] = v_pi
            o_ni[dst] = v_ni
            o_pp[dst] = v_pp
            o_np[dst] = v_np
        return 0

    lax.fori_loop(0, N_CH, chunk_body, 0)

    pltpu.sync_copy(o_pi, o_pi_hbm.at[pl.ds(base, PER_W)])
    pltpu.sync_copy(o_ni, o_ni_hbm.at[pl.ds(base, PER_W)])
    pltpu.sync_copy(o_pp, o_pp_hbm.at[pl.ds(base, PER_W)])
    pltpu.sync_copy(o_np, o_np_hbm.at[pl.ds(base, PER_W)])


_scores = pl.kernel(
    _score_body,
    out_type=tuple(jax.ShapeDtypeStruct((BL,), jnp.float32) for _ in range(4)),
    mesh=plsc.VectorSubcoreMesh(core_axis_name="c", subcore_axis_name="s"),
    compiler_params=pltpu.CompilerParams(needs_layout_passes=False, use_tc_tiling_on_sc=False),
    scratch_types=(
        [pltpu.VMEM((CHUNK,), jnp.int32) for _ in range(3)]
        + [pltpu.VMEM((CHUNK, D), jnp.float32) for _ in range(6)]
        + [pltpu.VMEM((PER_W,), jnp.float32) for _ in range(4)]
        + [pltpu.SemaphoreType.DMA]
    ),
)


LOSS_ROWS = BL // 128
LOSS_BLK = 128
LOSS_GRID = LOSS_ROWS // LOSS_BLK


def _loss_body(pi_ref, ni_ref, pp_ref, np_ref, m_ref, out_ref):
    i = pl.program_id(0)

    @pl.when(i == 0)
    def _():
        out_ref[0, 0] = 0.0
        out_ref[0, 1] = 0.0
        out_ref[0, 2] = 0.0

    pi = pi_ref[...]
    ni = ni_ref[...]
    pp = pp_ref[...]
    np_ = np_ref[...]
    m = m_ref[...]
    ls = jax.nn.log_sigmoid
    v_int = jnp.sum(m * ls(pi - ni))
    v_pop = jnp.sum(m * ls(np_ - pp) + (1.0 - m) * ls(pp - np_))
    v_tot = jnp.sum(ls((pi + pp) - (ni + np_)))
    out_ref[0, 0] += v_int
    out_ref[0, 1] += v_pop
    out_ref[0, 2] += v_tot


def _loss_tc(pi, ni, pp, np_, m):
    spec = pl.BlockSpec((LOSS_BLK, 128), lambda i: (i, 0))
    return pl.pallas_call(
        _loss_body,
        grid=(LOSS_GRID,),
        in_specs=[spec] * 5,
        out_specs=pl.BlockSpec(memory_space=pltpu.SMEM),
        out_shape=jax.ShapeDtypeStruct((1, 3), jnp.float32),
    )(pi, ni, pp, np_, m)


TAB_ROWS = NUM_ROWS * D // 128
TAB_BLK = 1000
TAB_GRID = TAB_ROWS // TAB_BLK


def _disc_body(ii_ref, ip_ref, ui_ref, up_ref, fi_ref, fu_ref, cs_ref,
               out_ref, acc_ref):
    i = pl.program_id(0)

    @pl.when(i == 0)
    def _():
        acc_ref[0] = 0.0
        acc_ref[1] = 0.0
        acc_ref[2] = 0.0
        acc_ref[3] = 0.0

    def term(a_ref, b_ref, f_ref):
        dlt = a_ref[...] - b_ref[...]
        sq = dlt * dlt
        rs = jnp.sum(sq.reshape(TAB_BLK, 8, 16), axis=2)
        f = f_ref[...] > 0.0
        return (jnp.sum(jnp.where(f, rs, 0.0)),
                jnp.sum(f.astype(jnp.float32)))

    ti, ci = term(ii_ref, ip_ref, fi_ref)
    tu, cu = term(ui_ref, up_ref, fu_ref)
    acc_ref[0] += ti
    acc_ref[1] += ci
    acc_ref[2] += tu
    acc_ref[3] += cu

    @pl.when(i == TAB_GRID - 1)
    def _():
        s_int = cs_ref[0, 0]
        s_pop = cs_ref[0, 1]
        s_tot = cs_ref[0, 2]
        inv = 1.0 / BL
        disc = acc_ref[0] / (acc_ref[1] * D) + acc_ref[2] / (acc_ref[3] * D)
        out_ref[0, 0] = -s_tot * inv
        out_ref[0, 1] = -INT_WEIGHT * s_int * inv
        out_ref[0, 2] = -POP_WEIGHT * s_pop * inv
        out_ref[0, 3] = -DIS_PEN * disc


def _disc_tc(ii, ip_, ui, up, fi, fu, csums):
    tspec = pl.BlockSpec((TAB_BLK, 128), lambda i: (i, 0))
    fspec = pl.BlockSpec((TAB_BLK, 8), lambda i: (i, 0))
    return pl.pallas_call(
        _disc_body,
        grid=(TAB_GRID,),
        in_specs=[tspec, tspec, tspec, tspec, fspec, fspec,
                  pl.BlockSpec(memory_space=pltpu.SMEM)],
        out_specs=pl.BlockSpec(memory_space=pltpu.SMEM),
        out_shape=jax.ShapeDtypeStruct((1, 4), jnp.float32),
        scratch_shapes=[pltpu.SMEM((4,), jnp.float32)],
    )(ii, ip_, ui, up, fi, fu, csums)


def kernel(user, item_p, item_n, mask, users_int, users_pop,
           items_int, items_pop):
    user_f = user.reshape(-1)
    itemp_f = item_p.reshape(-1)
    itemn_f = item_n.reshape(-1)
    mask_f = mask.reshape(-1).astype(jnp.float32)

    cnt_i2, cnt_u2 = _hist(itemp_f, itemn_f, user_f)
    flags_i = jnp.concatenate([cnt_i2[:HALF], cnt_i2[CNT_SZ:CNT_SZ + HALF]])
    flags_u = jnp.concatenate([cnt_u2[:HALF], cnt_u2[CNT_SZ:CNT_SZ + HALF]])

    spi, sni, spp, snp = _scores(user_f, itemp_f, itemn_f,
                                 users_int, users_pop, items_int, items_pop)

    csums = _loss_tc(spi.reshape(LOSS_ROWS, 128), sni.reshape(LOSS_ROWS, 128),
                     spp.reshape(LOSS_ROWS, 128), snp.reshape(LOSS_ROWS, 128),
                     mask_f.reshape(LOSS_ROWS, 128))

    out = _disc_tc(items_int.reshape(TAB_ROWS, 128),
                   items_pop.reshape(TAB_ROWS, 128),
                   users_int.reshape(TAB_ROWS, 128),
                   users_pop.reshape(TAB_ROWS, 128),
                   flags_i.reshape(TAB_ROWS, 8),
                   flags_u.reshape(TAB_ROWS, 8),
                   csums)
    return out.reshape(4)


# speedup vs baseline: 2.0000x; 2.0000x over previous
import jax
import jax.numpy as jnp
from jax import lax
from jax.experimental import pallas as pl
from jax.experimental.pallas import tpu as pltpu
from jax.experimental.pallas import tpu_sc as plsc

NUM_ROWS = 1_000_000
D = 16
B = 16384
L = 20
BL = B * L
INT_WEIGHT = 0.1
POP_WEIGHT = 0.1
DIS_PEN = 0.1

NC = 2
NS = 16
NW = NC * NS

HALF = NUM_ROWS // NC
SLAB = 31264
CNT_SZ = NS * SLAB
DUMP = HALF

PER_W = BL // NW
CHUNK = 128
N_CH = PER_W // CHUNK

IDX_BATCH = 4096
N_SCAT = IDX_BATCH // CHUNK
PER_S = BL // NS
N_BATCH = PER_S // IDX_BATCH

_SC_PARAMS = pltpu.CompilerParams(
    needs_layout_passes=False, use_tc_tiling_on_sc=False)


def _fill_const(ref, n, value, dtype):
    vec = jnp.full((16,), value, dtype=dtype)
    for k in range(n // 16):
        ref[pl.ds(k * 16, 16)] = vec


def _hist_body(itemp_hbm, itemn_hbm, user_hbm, cnt_i_out, cnt_u_out,
               idx_v, ones_v, zer_v, cnt_i_sp, cnt_u_sp, sem, *tmp_vs):
    cid = lax.axis_index("c")
    sid = lax.axis_index("s")
    lo = cid * HALF

    _fill_const(ones_v, CHUNK, 1.0, jnp.float32)

    def zero_body(i, _):
        zer_v[pl.ds(i * 16, 16)] = jnp.zeros((16,), jnp.float32)
        return 0

    lax.fori_loop(0, SLAB // 16, zero_body, 0)
    pltpu.sync_copy(zer_v, cnt_i_sp.at[pl.ds(sid * SLAB, SLAB)])
    pltpu.sync_copy(zer_v, cnt_u_sp.at[pl.ds(sid * SLAB, SLAB)])
    plsc.subcore_barrier()

    def scatter_all(src_hbm, cnt_sp):
        base = sid * PER_S

        def batch_body(bi, _):
            pltpu.sync_copy(src_hbm.at[pl.ds(base + bi * IDX_BATCH, IDX_BATCH)],
                            idx_v)
            cps = []
            for j in range(N_SCAT):
                tmp_v = tmp_vs[j]
                for k in range(CHUNK // 16):
                    v = idx_v[pl.ds(j * CHUNK + k * 16, 16)]
                    loc = v - lo
                    ok = (loc >= 0) & (loc < HALF)
                    tmp_v[pl.ds(k * 16, 16)] = jnp.where(ok, loc, DUMP)
                cps.append(pltpu.async_copy(ones_v, cnt_sp.at[tmp_v], sem,
                                            add=True))
            for cp in cps:
                cp.wait()
            return 0

        lax.fori_loop(0, N_BATCH, batch_body, 0)

    scatter_all(itemp_hbm, cnt_i_sp)
    scatter_all(itemn_hbm, cnt_i_sp)
    scatter_all(user_hbm, cnt_u_sp)
    plsc.subcore_barrier()

    pltpu.sync_copy(cnt_i_sp.at[pl.ds(sid * SLAB, SLAB)], zer_v)
    pltpu.sync_copy(zer_v, cnt_i_out.at[pl.ds(cid * CNT_SZ + sid * SLAB, SLAB)])
    pltpu.sync_copy(cnt_u_sp.at[pl.ds(sid * SLAB, SLAB)], zer_v)
    pltpu.sync_copy(zer_v, cnt_u_out.at[pl.ds(cid * CNT_SZ + sid * SLAB, SLAB)])


_hist = pl.kernel(
    _hist_body,
    out_type=(jax.ShapeDtypeStruct((NC * CNT_SZ,), jnp.float32),
              jax.ShapeDtypeStruct((NC * CNT_SZ,), jnp.float32)),
    mesh=plsc.VectorSubcoreMesh(core_axis_name="c", subcore_axis_name="s"),
    compiler_params=_SC_PARAMS,
    scratch_types=(
        [pltpu.VMEM((IDX_BATCH,), jnp.int32),
         pltpu.VMEM((CHUNK,), jnp.float32),
         pltpu.VMEM((SLAB,), jnp.float32),
         pltpu.VMEM_SHARED((CNT_SZ,), jnp.float32),
         pltpu.VMEM_SHARED((CNT_SZ,), jnp.float32),
         pltpu.SemaphoreType.DMA]
        + [pltpu.VMEM((CHUNK,), jnp.int32) for _ in range(N_SCAT)]
    ),
)


def _score_body(user_hbm, itemp_hbm, itemn_hbm,
                u_int_hbm, u_pop_hbm, i_int_hbm, i_pop_hbm,
                cnt_i_hbm, cnt_u_hbm,
                o_pi_hbm, o_ni_hbm, o_pp_hbm, o_np_hbm, disc_hbm,
                xu_v, xp_v, xn_v,
                o_pi, o_ni, o_pp, o_np, dpart,
                rows0, rows1, cnts0, cnts1, adjs0, adjs1, sems):
    cid = lax.axis_index("c")
    sid = lax.axis_index("s")
    wid = sid * NC + cid
    base = wid * PER_W

    pltpu.sync_copy(user_hbm.at[pl.ds(base, PER_W)], xu_v)
    pltpu.sync_copy(itemp_hbm.at[pl.ds(base, PER_W)], xp_v)
    pltpu.sync_copy(itemn_hbm.at[pl.ds(base, PER_W)], xn_v)

    rows = (rows0, rows1)
    cnts = (cnts0, cnts1)
    adjs = (adjs0, adjs1)

    def adj_pos(v):
        return jnp.where(v < HALF, v, v - HALF + CNT_SZ)

    def fire(g, b):
        sl = pl.ds(g * CHUNK, CHUNK)
        ru_i, ru_p, rp_i, rp_p, rn_i, rn_p = rows[b]
        cu, cp_, cn = cnts[b]
        au, ap_, an = adjs[b]
        sem = sems.at[b]
        pltpu.async_copy(u_int_hbm.at[xu_v.at[sl]], ru_i, sem)
        pltpu.async_copy(u_pop_hbm.at[xu_v.at[sl]], ru_p, sem)
        pltpu.async_copy(i_int_hbm.at[xp_v.at[sl]], rp_i, sem)
        pltpu.async_copy(i_pop_hbm.at[xp_v.at[sl]], rp_p, sem)
        pltpu.async_copy(i_int_hbm.at[xn_v.at[sl]], rn_i, sem)
        pltpu.async_copy(i_pop_hbm.at[xn_v.at[sl]], rn_p, sem)
        for k in range(CHUNK // 16):
            s16 = pl.ds(g * CHUNK + k * 16, 16)
            d16 = pl.ds(k * 16, 16)
            au[d16] = adj_pos(xu_v[s16])
            ap_[d16] = adj_pos(xp_v[s16])
            an[d16] = adj_pos(xn_v[s16])
        pltpu.async_copy(cnt_u_hbm.at[au], cu, sem)
        pltpu.async_copy(cnt_i_hbm.at[ap_], cp_, sem)
        pltpu.async_copy(cnt_i_hbm.at[an], cn, sem)

    def drain(b):
        ru_i, ru_p, rp_i, rp_p, rn_i, rn_p = rows[b]
        cu, cp_, cn = cnts[b]
        sem = sems.at[b]
        h = u_int_hbm.at[pl.ds(0, CHUNK), :]
        h1 = cnt_u_hbm.at[pl.ds(0, CHUNK)]
        for dst in (ru_i, ru_p, rp_i, rp_p, rn_i, rn_p):
            pltpu.make_async_copy(h, dst, sem).wait()
        for dst in (cu, cp_, cn):
            pltpu.make_async_copy(h1, dst, sem).wait()

    def compute(g, b, carry):
        acc_ti, acc_ci, acc_tu, acc_cu = carry
        ru_i, ru_p, rp_i, rp_p, rn_i, rn_p = rows[b]
        cu, cp_, cn = cnts[b]
        for grp in range(CHUNK // 16):
            rows16 = grp * 16 + lax.iota(jnp.int32, 16)
            inv_u = 1.0 / cu[pl.ds(grp * 16, 16)]
            inv_p = 1.0 / cp_[pl.ds(grp * 16, 16)]
            inv_n = 1.0 / cn[pl.ds(grp * 16, 16)]
            v_pi = jnp.zeros((16,), jnp.float32)
            v_ni = jnp.zeros((16,), jnp.float32)
            v_pp = jnp.zeros((16,), jnp.float32)
            v_np = jnp.zeros((16,), jnp.float32)
            for d in range(D):
                dd = jnp.full((16,), d, jnp.int32)
                cu_i = plsc.load_gather(ru_i, [rows16, dd])
                cu_p = plsc.load_gather(ru_p, [rows16, dd])
                cp_i = plsc.load_gather(rp_i, [rows16, dd])
                cp_p = plsc.load_gather(rp_p, [rows16, dd])
                cn_i = plsc.load_gather(rn_i, [rows16, dd])
                cn_p = plsc.load_gather(rn_p, [rows16, dd])
                v_pi = v_pi + cu_i * cp_i
                v_ni = v_ni + cu_i * cn_i
                v_pp = v_pp + cu_p * cp_p
                v_np = v_np + cu_p * cn_p
                du = cu_i - cu_p
                dp = cp_i - cp_p
                dn = cn_i - cn_p
                acc_ti = acc_ti + (dp * dp) * inv_p + (dn * dn) * inv_n
                acc_tu = acc_tu + (du * du) * inv_u
            acc_ci = acc_ci + inv_p + inv_n
            acc_cu = acc_cu + inv_u
            dst = pl.ds(g * CHUNK + grp * 16, 16)
            o_pi[dst] = v_pi
            o_ni[dst] = v_ni
            o_pp[dst] = v_pp
            o_np[dst] = v_np
        return acc_ti, acc_ci, acc_tu, acc_cu

    zero4 = tuple(jnp.zeros((16,), jnp.float32) for _ in range(4))
    fire(0, 0)

    def pair_body(t, carry):
        fire(2 * t + 1, 1)
        drain(0)
        carry = compute(2 * t, 0, carry)

        @pl.when(t < N_CH // 2 - 1)
        def _():
            fire(2 * t + 2, 0)

        drain(1)
        carry = compute(2 * t + 1, 1, carry)
        return carry

    acc_ti, acc_ci, acc_tu, acc_cu = lax.fori_loop(
        0, N_CH // 2, pair_body, zero4)

    dpart[pl.ds(0, 16)] = acc_ti
    dpart[pl.ds(16, 16)] = acc_ci
    dpart[pl.ds(32, 16)] = acc_tu
    dpart[pl.ds(48, 16)] = acc_cu
    pltpu.sync_copy(dpart, disc_hbm.at[pl.ds(wid * 64, 64)])

    pltpu.sync_copy(o_pi, o_pi_hbm.at[pl.ds(base, PER_W)])
    pltpu.sync_copy(o_ni, o_ni_hbm.at[pl.ds(base, PER_W)])
    pltpu.sync_copy(o_pp, o_pp_hbm.at[pl.ds(base, PER_W)])
    pltpu.sync_copy(o_np, o_np_hbm.at[pl.ds(base, PER_W)])


def _mk_rowbufs():
    return tuple(pltpu.VMEM((CHUNK, D), jnp.float32) for _ in range(6))


_scores = pl.kernel(
    _score_body,
    out_type=(tuple(jax.ShapeDtypeStruct((BL,), jnp.float32)
                    for _ in range(4))
              + (jax.ShapeDtypeStruct((NW * 64,), jnp.float32),)),
    mesh=plsc.VectorSubcoreMesh(core_axis_name="c", subcore_axis_name="s"),
    compiler_params=_SC_PARAMS,
    scratch_types=(
        [pltpu.VMEM((PER_W,), jnp.int32) for _ in range(3)]
        + [pltpu.VMEM((PER_W,), jnp.float32) for _ in range(4)]
        + [pltpu.VMEM((64,), jnp.float32)]
        + [_mk_rowbufs(), _mk_rowbufs()]
        + [tuple(pltpu.VMEM((CHUNK,), jnp.float32) for _ in range(3)),
           tuple(pltpu.VMEM((CHUNK,), jnp.float32) for _ in range(3))]
        + [tuple(pltpu.VMEM((CHUNK,), jnp.int32) for _ in range(3)),
           tuple(pltpu.VMEM((CHUNK,), jnp.int32) for _ in range(3))]
        + [pltpu.SemaphoreType.DMA((2,))]
    ),
)


LOSS_ROWS = BL // 128
LOSS_BLK = 128
LOSS_GRID = LOSS_ROWS // LOSS_BLK

DP_ROWS = NW * 64 // 128


def _loss_body(pi_ref, ni_ref, pp_ref, np_ref, m_ref, dp_ref, out_ref,
               acc_ref):
    i = pl.program_id(0)

    @pl.when(i == 0)
    def _():
        acc_ref[0] = 0.0
        acc_ref[1] = 0.0
        acc_ref[2] = 0.0

    pi = pi_ref[...]
    ni = ni_ref[...]
    pp = pp_ref[...]
    np_ = np_ref[...]
    m = m_ref[...]
    ls = jax.nn.log_sigmoid
    acc_ref[0] += jnp.sum(m * ls(pi - ni))
    acc_ref[1] += jnp.sum(m * ls(np_ - pp) + (1.0 - m) * ls(pp - np_))
    acc_ref[2] += jnp.sum(ls((pi + pp) - (ni + np_)))

    @pl.when(i == LOSS_GRID - 1)
    def _():
        dp = dp_ref[...]
        cidx = lax.broadcasted_iota(jnp.int32, (DP_ROWS, 128), 1) % 64
        tot_i = jnp.sum(jnp.where(cidx < 16, dp, 0.0))
        cnt_i = jnp.sum(jnp.where((cidx >= 16) & (cidx < 32), dp, 0.0))
        tot_u = jnp.sum(jnp.where((cidx >= 32) & (cidx < 48), dp, 0.0))
        cnt_u = jnp.sum(jnp.where(cidx >= 48, dp, 0.0))
        disc = tot_i / (cnt_i * D) + tot_u / (cnt_u * D)
        inv = 1.0 / BL
        out_ref[0, 0] = -acc_ref[2] * inv
        out_ref[0, 1] = -INT_WEIGHT * acc_ref[0] * inv
        out_ref[0, 2] = -POP_WEIGHT * acc_ref[1] * inv
        out_ref[0, 3] = -DIS_PEN * disc


def _loss_tc(pi, ni, pp, np_, m, dparts):
    spec = pl.BlockSpec((LOSS_BLK, 128), lambda i: (i, 0))
    dspec = pl.BlockSpec((DP_ROWS, 128), lambda i: (0, 0))
    return pl.pallas_call(
        _loss_body,
        grid=(LOSS_GRID,),
        in_specs=[spec] * 5 + [dspec],
        out_specs=pl.BlockSpec(memory_space=pltpu.SMEM),
        out_shape=jax.ShapeDtypeStruct((1, 4), jnp.float32),
        scratch_shapes=[pltpu.SMEM((3,), jnp.float32)],
    )(pi, ni, pp, np_, m, dparts)


def kernel(user, item_p, item_n, mask, users_int, users_pop,
           items_int, items_pop):
    user_f = user.reshape(-1)
    itemp_f = item_p.reshape(-1)
    itemn_f = item_n.reshape(-1)
    mask_f = mask.reshape(-1).astype(jnp.float32)

    cnt_i, cnt_u = _hist(itemp_f, itemn_f, user_f)

    spi, sni, spp, snp, dparts = _scores(
        user_f, itemp_f, itemn_f,
        users_int, users_pop, items_int, items_pop, cnt_i, cnt_u)

    out = _loss_tc(spi.reshape(LOSS_ROWS, 128), sni.reshape(LOSS_ROWS, 128),
                   spp.reshape(LOSS_ROWS, 128), snp.reshape(LOSS_ROWS, 128),
                   mask_f.reshape(LOSS_ROWS, 128),
                   dparts.reshape(DP_ROWS, 128))
    return out.reshape(4)
